# baseline (speedup 1.0000x reference)
import functools

import jax
import jax.numpy as jnp
from jax import lax
from jax.experimental import pallas as pl
from jax.experimental.pallas import tpu as pltpu
from jax.experimental.pallas import tpu_sc as plsc

_B = 16384
_D = 128
_NC = 2
_NS = 16
_NW = _NC * _NS
_BPW = _B // _NW
_CHUNK = 128
_NCH = _BPW // _CHUNK


def _body(h_hbm, dom_hbm, emb_hbm, out_hbm,
          idx_v, hbufs, sem_i, sem_h, sem_e, sem_o):
    wid = lax.axis_index("s") * _NC + lax.axis_index("c")
    base = wid * _BPW

    idx_cp = pltpu.make_async_copy(dom_hbm.at[pl.ds(base, _BPW)], idx_v, sem_i)
    idx_cp.start()
    for ci in range(_NCH):
        pltpu.make_async_copy(
            h_hbm.at[pl.ds(base + ci * _CHUNK, _CHUNK)],
            hbufs[ci], sem_h.at[ci]).start()
    idx_cp.wait()

    gadds = []
    for ci in range(_NCH):
        pltpu.make_async_copy(
            h_hbm.at[pl.ds(base + ci * _CHUNK, _CHUNK)],
            hbufs[ci], sem_h.at[ci]).wait()
        cp = pltpu.async_copy(
            emb_hbm.at[idx_v.at[pl.ds(ci * _CHUNK, _CHUNK)]],
            hbufs[ci], sem_e.at[ci], add=True)
        gadds.append(cp)

    wbs = []
    for ci in range(_NCH):
        gadds[ci].wait()
        cp = pltpu.make_async_copy(
            hbufs[ci], out_hbm.at[pl.ds(base + ci * _CHUNK, _CHUNK)],
            sem_o.at[ci])
        cp.start()
        wbs.append(cp)

    for ci in range(_NCH):
        wbs[ci].wait()


@jax.jit
def _domain_token(h, domain, emb):
    mesh = plsc.VectorSubcoreMesh(core_axis_name="c", subcore_axis_name="s")
    return pl.kernel(
        _body,
        out_type=jax.ShapeDtypeStruct((_B, _D), jnp.float32),
        mesh=mesh,
        scratch_types=[
            pltpu.VMEM((_BPW,), jnp.int32),
            [pltpu.VMEM((_CHUNK, _D), jnp.float32) for _ in range(_NCH)],
            pltpu.SemaphoreType.DMA,
            pltpu.SemaphoreType.DMA((_NCH,)),
            pltpu.SemaphoreType.DMA((_NCH,)),
            pltpu.SemaphoreType.DMA((_NCH,)),
        ],
    )(h, domain, emb)


def kernel(h, domain, emb):
    return _domain_token(h, domain.astype(jnp.int32), emb)
